# Initial kernel scaffold; baseline (speedup 1.0000x reference)
import jax
import jax.numpy as jnp
from jax.experimental import pallas as pl


def kernel(x, adj, W, a):
    raise NotImplementedError("write your pallas kernel here")


# speedup vs baseline: 842.6043x; 842.6043x over previous
import functools

import jax
import jax.numpy as jnp
from jax.experimental import pallas as pl

N = 1024
IN_F = 128
OUT_F = 64
BLOCK_ROWS = 256
NEG_BIG = -9000000000000000.0


def _gat_kernel(x_ref, xb_ref, adj_ref, w_ref, a_ref, out_ref):
    h = jnp.dot(x_ref[...], w_ref[...], preferred_element_type=jnp.float32)
    a_vec = a_ref[...]
    g_full = jnp.dot(h, a_vec[OUT_F:, :], preferred_element_type=jnp.float32)
    h_blk = jnp.dot(xb_ref[...], w_ref[...], preferred_element_type=jnp.float32)
    f_blk = jnp.dot(h_blk, a_vec[:OUT_F, :], preferred_element_type=jnp.float32)
    s = f_blk + g_full.reshape(1, N)
    s = jnp.where(s >= 0, s, 0.2 * s)
    logits = jnp.where(adj_ref[...] > 0, s, NEG_BIG)
    m = jnp.max(logits, axis=1, keepdims=True)
    p = jnp.exp(logits - m)
    att = p / jnp.sum(p, axis=1, keepdims=True)
    o = jnp.dot(att, h, preferred_element_type=jnp.float32)
    out_ref[...] = jnp.where(o > 0, o, jnp.exp(o) - 1.0)


@jax.jit
def kernel(x, adj, W, a):
    grid = (N // BLOCK_ROWS,)
    return pl.pallas_call(
        _gat_kernel,
        grid=grid,
        in_specs=[
            pl.BlockSpec((N, IN_F), lambda i: (0, 0)),
            pl.BlockSpec((BLOCK_ROWS, IN_F), lambda i: (i, 0)),
            pl.BlockSpec((BLOCK_ROWS, N), lambda i: (i, 0)),
            pl.BlockSpec((IN_F, OUT_F), lambda i: (0, 0)),
            pl.BlockSpec((2 * OUT_F, 1), lambda i: (0, 0)),
        ],
        out_specs=pl.BlockSpec((BLOCK_ROWS, OUT_F), lambda i: (i, 0)),
        out_shape=jax.ShapeDtypeStruct((N, OUT_F), jnp.float32),
    )(x, x, adj, W, a)


# speedup vs baseline: 868.2523x; 1.0304x over previous
import functools

import jax
import jax.numpy as jnp
from jax.experimental import pallas as pl

N = 1024
IN_F = 128
OUT_F = 64
BLOCK_ROWS = 256
NEG_BIG = -9000000000000000.0


def _gat_kernel(x_ref, xb_ref, adj_ref, w_ref, a_ref, out_ref):
    h = jnp.dot(x_ref[...], w_ref[...], preferred_element_type=jnp.float32)
    a_vec = a_ref[...]
    g_full = jnp.dot(h, a_vec[OUT_F:, :], preferred_element_type=jnp.float32)
    h_blk = jnp.dot(xb_ref[...], w_ref[...], preferred_element_type=jnp.float32)
    f_blk = jnp.dot(h_blk, a_vec[:OUT_F, :], preferred_element_type=jnp.float32)
    s = f_blk + g_full.reshape(1, N)
    s = jnp.maximum(s, 0.2 * s)
    logits = jnp.where(adj_ref[...] > 0, s, NEG_BIG)
    m = jnp.max(logits, axis=1, keepdims=True)
    p = jnp.exp(logits - m)
    denom = jnp.sum(p, axis=1, keepdims=True)
    o = jnp.dot(p, h, preferred_element_type=jnp.float32) / denom
    out_ref[...] = jnp.where(o > 0, o, jnp.exp(o) - 1.0)


@jax.jit
def kernel(x, adj, W, a):
    grid = (N // BLOCK_ROWS,)
    return pl.pallas_call(
        _gat_kernel,
        grid=grid,
        in_specs=[
            pl.BlockSpec((N, IN_F), lambda i: (0, 0)),
            pl.BlockSpec((BLOCK_ROWS, IN_F), lambda i: (i, 0)),
            pl.BlockSpec((BLOCK_ROWS, N), lambda i: (i, 0)),
            pl.BlockSpec((IN_F, OUT_F), lambda i: (0, 0)),
            pl.BlockSpec((2 * OUT_F, 1), lambda i: (0, 0)),
        ],
        out_specs=pl.BlockSpec((BLOCK_ROWS, OUT_F), lambda i: (i, 0)),
        out_shape=jax.ShapeDtypeStruct((N, OUT_F), jnp.float32),
    )(x, x, adj, W, a)


# speedup vs baseline: 882.5827x; 1.0165x over previous
import functools

import jax
import jax.numpy as jnp
from jax.experimental import pallas as pl

N = 1024
IN_F = 128
OUT_F = 64
BLOCK_ROWS = 256
NEG_BIG = -9000000000000000.0


def _gat_kernel(x_ref, xb_ref, adj_ref, w_ref, a_ref, out_ref):
    h = jnp.dot(x_ref[...], w_ref[...], preferred_element_type=jnp.float32)
    a_vec = a_ref[...]
    g_full = jnp.dot(h, a_vec[OUT_F:, :], preferred_element_type=jnp.float32)
    h_blk = jnp.dot(xb_ref[...], w_ref[...], preferred_element_type=jnp.float32)
    f_blk = jnp.dot(h_blk, a_vec[:OUT_F, :], preferred_element_type=jnp.float32)
    adj = adj_ref[...]
    fg = f_blk + jnp.max(g_full)
    mhat = jnp.maximum(fg, 0.2 * fg)
    s = f_blk + g_full.reshape(1, N)
    s = jnp.maximum(s, 0.2 * s)
    p = adj * jnp.exp(s - mhat)
    ones = jnp.ones((N, 1), dtype=jnp.float32)
    row_cnt = jnp.dot(adj, ones, preferred_element_type=jnp.float32)
    denom = jnp.maximum(jnp.sum(p, axis=1, keepdims=True), 1e-38)
    o = jnp.dot(p, h, preferred_element_type=jnp.float32) / denom
    hmean = jnp.sum(h, axis=0, keepdims=True) * (1.0 / N)
    o = jnp.where(row_cnt > 0, o, hmean)
    out_ref[...] = jnp.where(o > 0, o, jnp.exp(o) - 1.0)


@jax.jit
def kernel(x, adj, W, a):
    grid = (N // BLOCK_ROWS,)
    return pl.pallas_call(
        _gat_kernel,
        grid=grid,
        in_specs=[
            pl.BlockSpec((N, IN_F), lambda i: (0, 0)),
            pl.BlockSpec((BLOCK_ROWS, IN_F), lambda i: (i, 0)),
            pl.BlockSpec((BLOCK_ROWS, N), lambda i: (i, 0)),
            pl.BlockSpec((IN_F, OUT_F), lambda i: (0, 0)),
            pl.BlockSpec((2 * OUT_F, 1), lambda i: (0, 0)),
        ],
        out_specs=pl.BlockSpec((BLOCK_ROWS, OUT_F), lambda i: (i, 0)),
        out_shape=jax.ShapeDtypeStruct((N, OUT_F), jnp.float32),
    )(x, x, adj, W, a)


# speedup vs baseline: 980.0229x; 1.1104x over previous
import functools

import jax
import jax.numpy as jnp
from jax.experimental import pallas as pl

N = 1024
IN_F = 128
OUT_F = 64
BLOCK_ROWS = 1024
NEG_BIG = -9000000000000000.0


def _gat_kernel(x_ref, xb_ref, adj_ref, w_ref, a_ref, out_ref):
    h = jnp.dot(x_ref[...], w_ref[...], preferred_element_type=jnp.float32)
    a_vec = a_ref[...]
    g_full = jnp.dot(h, a_vec[OUT_F:, :], preferred_element_type=jnp.float32)
    h_blk = jnp.dot(xb_ref[...], w_ref[...], preferred_element_type=jnp.float32)
    f_blk = jnp.dot(h_blk, a_vec[:OUT_F, :], preferred_element_type=jnp.float32)
    adj = adj_ref[...]
    fg = f_blk + jnp.max(g_full)
    mhat = jnp.maximum(fg, 0.2 * fg)
    s = f_blk + g_full.reshape(1, N)
    s = jnp.maximum(s, 0.2 * s)
    p = adj * jnp.exp(s - mhat)
    ones = jnp.ones((N, 1), dtype=jnp.float32)
    row_cnt = jnp.dot(adj, ones, preferred_element_type=jnp.float32)
    denom = jnp.maximum(jnp.sum(p, axis=1, keepdims=True), 1e-38)
    o = jnp.dot(p, h, preferred_element_type=jnp.float32) / denom
    hmean = jnp.sum(h, axis=0, keepdims=True) * (1.0 / N)
    o = jnp.where(row_cnt > 0, o, hmean)
    out_ref[...] = jnp.where(o > 0, o, jnp.exp(o) - 1.0)


@jax.jit
def kernel(x, adj, W, a):
    grid = (N // BLOCK_ROWS,)
    return pl.pallas_call(
        _gat_kernel,
        grid=grid,
        in_specs=[
            pl.BlockSpec((N, IN_F), lambda i: (0, 0)),
            pl.BlockSpec((BLOCK_ROWS, IN_F), lambda i: (i, 0)),
            pl.BlockSpec((BLOCK_ROWS, N), lambda i: (i, 0)),
            pl.BlockSpec((IN_F, OUT_F), lambda i: (0, 0)),
            pl.BlockSpec((2 * OUT_F, 1), lambda i: (0, 0)),
        ],
        out_specs=pl.BlockSpec((BLOCK_ROWS, OUT_F), lambda i: (i, 0)),
        out_shape=jax.ShapeDtypeStruct((N, OUT_F), jnp.float32),
    )(x, x, adj, W, a)


# speedup vs baseline: 1032.4588x; 1.0535x over previous
import jax
import jax.numpy as jnp
from jax.experimental import pallas as pl

N = 1024
IN_F = 128
OUT_F = 64
LOG2E = 1.4426950408889634


def _gat_kernel(x_ref, adj_ref, w_ref, a_ref, out_ref):
    h = jnp.dot(x_ref[...], w_ref[...], preferred_element_type=jnp.float32)
    a_vec = a_ref[...]
    f = jnp.dot(h, a_vec[:OUT_F, :], preferred_element_type=jnp.float32)
    g = jnp.dot(h, a_vec[OUT_F:, :], preferred_element_type=jnp.float32)
    fg = f + jnp.max(g)
    mhat = jnp.maximum(fg, 0.2 * fg)
    u = (f - mhat) * LOG2E
    v = (0.2 * f - mhat) * LOG2E
    g_row = g.reshape(1, N) * LOG2E
    g2_row = 0.2 * g_row
    e2 = jnp.maximum(u + g_row, v + g2_row)
    p = adj_ref[...] * jnp.exp2(e2)
    ones = jnp.ones((N, 1), dtype=jnp.float32)
    h_ext = jnp.concatenate([h, ones], axis=1)
    o_ext = jnp.dot(p, h_ext, preferred_element_type=jnp.float32)
    denom = o_ext[:, OUT_F:]
    o = o_ext[:, :OUT_F] / denom
    hmean = jnp.sum(h, axis=0, keepdims=True) * (1.0 / N)
    o = jnp.where(denom > 0, o, hmean)
    out_ref[...] = jnp.where(o > 0, o, jnp.exp(o) - 1.0)


@jax.jit
def kernel(x, adj, W, a):
    return pl.pallas_call(
        _gat_kernel,
        out_shape=jax.ShapeDtypeStruct((N, OUT_F), jnp.float32),
    )(x, adj, W, a)
